# Initial kernel scaffold; baseline (speedup 1.0000x reference)
import jax
import jax.numpy as jnp
from jax.experimental import pallas as pl


def kernel(x):
    raise NotImplementedError("write your pallas kernel here")


# speedup vs baseline: 15.4250x; 15.4250x over previous
import functools

import numpy as np
import jax
import jax.numpy as jnp
from jax import lax
from jax.experimental import pallas as pl
from jax.experimental.pallas import tpu as pltpu

_L = 8192
_N1 = 64
_N2 = 128
_K2 = 72
_KMAX = 4096
_DT = 128


@functools.lru_cache(maxsize=None)
def _dft_consts():
    n1 = np.arange(_N1, dtype=np.float64)
    k1 = np.arange(_N1, dtype=np.float64)
    a1 = 2.0 * np.pi * np.outer(k1, n1) / _N1
    c64, s64 = np.cos(a1), -np.sin(a1)
    n2 = np.arange(_N2, dtype=np.float64)
    k2 = np.arange(_K2, dtype=np.float64)
    a2 = 2.0 * np.pi * np.outer(k2, n2) / _N2
    c128, s128 = np.cos(a2), -np.sin(a2)
    at = 2.0 * np.pi * np.outer(k1, n2) / _L
    twc, tws = np.cos(at), -np.sin(at)
    return tuple(np.asarray(v, dtype=np.float32)
                 for v in (c64, s64, c128, s128, twc, tws))


def _body(x_ref, c64_ref, s64_ref, c128_ref, s128_ref, twc_ref, tws_ref,
          seas_ref, trend_ref):
    f32 = jnp.float32
    x = x_ref[0]
    dn1 = (((1,), (0,)), ((), ()))
    dot = functools.partial(lax.dot_general, preferred_element_type=f32,
                            precision=lax.Precision.HIGHEST)
    are = dot(c64_ref[...], x, dn1)
    aim = dot(s64_ref[...], x, dn1)
    twc = twc_ref[...][:, :, None]
    tws = tws_ref[...][:, :, None]
    zre = are * twc - aim * tws
    zim = are * tws + aim * twc
    dn2 = (((1,), (1,)), ((), ()))
    xre = dot(c128_ref[...], zre, dn2) - dot(s128_ref[...], zim, dn2)
    xim = dot(c128_ref[...], zim, dn2) + dot(s128_ref[...], zre, dn2)
    m2 = xre * xre + xim * xim
    kv = (lax.broadcasted_iota(jnp.int32, (_K2, _N1), 0) * _N1
          + lax.broadcasted_iota(jnp.int32, (_K2, _N1), 1))[:, :, None]
    m2 = jnp.where((kv >= 1) & (kv <= _KMAX), m2, -1.0)

    n1i = lax.broadcasted_iota(jnp.int32, (_N1, 1), 0)
    n2i = lax.broadcasted_iota(jnp.int32, (_N2, 1), 0)
    seas = jnp.zeros_like(x)
    for _ in range(3):
        m = m2.max(axis=0).max(axis=0)
        is_max = m2 == m[None, None, :]
        kj = jnp.where(is_max, kv, _L).min(axis=0).min(axis=0)
        sel = kv == kj[None, None, :]
        re = jnp.where(sel, xre, 0.0).sum(axis=0).sum(axis=0)
        im = jnp.where(sel, xim, 0.0).sum(axis=0).sum(axis=0)
        m2 = jnp.where(sel, -1.0, m2)
        scale = jnp.where(kj == _KMAX, 1.0, 2.0) * (1.0 / _L)
        wre = re * scale
        wim = im * scale
        pa = (n1i * kj[None, :]) & (_N1 - 1)
        aa = pa.astype(f32) * f32(2.0 * np.pi / _N1)
        ca, sa = jnp.cos(aa), jnp.sin(aa)
        pb = (n2i * kj[None, :]) & (_L - 1)
        ab = pb.astype(f32) * f32(2.0 * np.pi / _L)
        cb, sb = jnp.cos(ab), jnp.sin(ab)
        u = wre[None, :] * ca - wim[None, :] * sa
        v = wre[None, :] * sa + wim[None, :] * ca
        seas = (seas + u[:, None, :] * cb[None, :, :]
                - v[:, None, :] * sb[None, :, :])
    seas_ref[0] = seas
    trend_ref[0] = x - seas


def _run(xr, interpret=False):
    B, n1, n2, D = xr.shape
    consts = [jnp.asarray(c) for c in _dft_consts()]
    grid = (B, D // _DT)
    cspec = lambda shape: pl.BlockSpec(shape, lambda b, j: (0, 0))
    out = pl.pallas_call(
        _body,
        grid=grid,
        in_specs=[
            pl.BlockSpec((1, _N1, _N2, _DT), lambda b, j: (b, 0, 0, j)),
            cspec((_N1, _N1)), cspec((_N1, _N1)),
            cspec((_K2, _N2)), cspec((_K2, _N2)),
            cspec((_N1, _N2)), cspec((_N1, _N2)),
        ],
        out_specs=[
            pl.BlockSpec((1, _N1, _N2, _DT), lambda b, j: (b, 0, 0, j)),
            pl.BlockSpec((1, _N1, _N2, _DT), lambda b, j: (b, 0, 0, j)),
        ],
        out_shape=[
            jax.ShapeDtypeStruct((B, _N1, _N2, D), jnp.float32),
            jax.ShapeDtypeStruct((B, _N1, _N2, D), jnp.float32),
        ],
        interpret=interpret,
    )(xr, *consts)
    return out


def kernel(x):
    B, L, D = x.shape
    xr = x.reshape(B, _N1, _N2, D)
    seas, trend = _run(xr)
    return (seas.reshape(B, L, D), trend.reshape(B, L, D))


# speedup vs baseline: 16.7735x; 1.0874x over previous
import functools

import numpy as np
import jax
import jax.numpy as jnp
from jax import lax
from jax.experimental import pallas as pl
from jax.experimental.pallas import tpu as pltpu

_L = 8192
_N1 = 64
_N2 = 128
_K2 = 72
_KMAX = 4096
_DT = 128


@functools.lru_cache(maxsize=None)
def _dft_consts():
    n1 = np.arange(_N1, dtype=np.float64)
    k1 = np.arange(_N1, dtype=np.float64)
    a1 = 2.0 * np.pi * np.outer(k1, n1) / _N1
    c64, s64 = np.cos(a1), -np.sin(a1)
    n2 = np.arange(_N2, dtype=np.float64)
    k2 = np.arange(_K2, dtype=np.float64)
    a2 = 2.0 * np.pi * np.outer(k2, n2) / _N2
    c128, s128 = np.cos(a2), -np.sin(a2)
    at = 2.0 * np.pi * np.outer(k1, n2) / _L
    twc, tws = np.cos(at), -np.sin(at)
    cs64 = np.vstack([c64, s64])
    cs128 = np.vstack([c128, s128])
    return tuple(np.asarray(v, dtype=np.float32)
                 for v in (cs64, cs128, twc, tws))


def _body(x_ref, cs64_ref, cs128_ref, twc_ref, tws_ref,
          seas_ref, trend_ref):
    f32 = jnp.float32
    x = x_ref[0]
    dn1 = (((1,), (0,)), ((), ()))
    dot = functools.partial(lax.dot_general, preferred_element_type=f32,
                            precision=lax.Precision.HIGHEST)
    a2 = dot(cs64_ref[...], x, dn1)
    are, aim = a2[:_N1], a2[_N1:]
    twc = twc_ref[...][:, :, None]
    tws = tws_ref[...][:, :, None]
    zre = are * twc - aim * tws
    zim = are * tws + aim * twc
    dn2 = (((1,), (1,)), ((), ()))
    p = dot(cs128_ref[...], zre, dn2)
    q = dot(cs128_ref[...], zim, dn2)
    xre = p[:_K2] - q[_K2:]
    xim = q[:_K2] + p[_K2:]
    m2 = xre * xre + xim * xim
    kv = (lax.broadcasted_iota(jnp.int32, (_K2, _N1), 0) * _N1
          + lax.broadcasted_iota(jnp.int32, (_K2, _N1), 1))[:, :, None]
    m2 = jnp.where((kv >= 1) & (kv <= _KMAX), m2, -1.0)

    n1i = lax.broadcasted_iota(jnp.int32, (_N1, 1), 0)
    n2i = lax.broadcasted_iota(jnp.int32, (_N2, 1), 0)
    seas = jnp.zeros_like(x)
    for _ in range(3):
        m = m2.max(axis=0).max(axis=0)
        is_max = m2 == m[None, None, :]
        kj = jnp.where(is_max, kv, _L).min(axis=0).min(axis=0)
        sel = kv == kj[None, None, :]
        re = jnp.where(sel, xre, 0.0).sum(axis=0).sum(axis=0)
        im = jnp.where(sel, xim, 0.0).sum(axis=0).sum(axis=0)
        m2 = jnp.where(sel, -1.0, m2)
        scale = jnp.where(kj == _KMAX, 1.0, 2.0) * (1.0 / _L)
        wre = re * scale
        wim = im * scale
        pa = (n1i * kj[None, :]) & (_N1 - 1)
        aa = pa.astype(f32) * f32(2.0 * np.pi / _N1)
        ca, sa = jnp.cos(aa), jnp.sin(aa)
        pb = (n2i * kj[None, :]) & (_L - 1)
        ab = pb.astype(f32) * f32(2.0 * np.pi / _L)
        cb, sb = jnp.cos(ab), jnp.sin(ab)
        u = wre[None, :] * ca - wim[None, :] * sa
        v = wre[None, :] * sa + wim[None, :] * ca
        seas = (seas + u[:, None, :] * cb[None, :, :]
                - v[:, None, :] * sb[None, :, :])
    seas_ref[0] = seas
    trend_ref[0] = x - seas


def _run(xr, interpret=False):
    B, n1, n2, D = xr.shape
    consts = [jnp.asarray(c) for c in _dft_consts()]
    grid = (B, D // _DT)
    cspec = lambda shape: pl.BlockSpec(shape, lambda b, j: (0, 0))
    out = pl.pallas_call(
        _body,
        grid=grid,
        in_specs=[
            pl.BlockSpec((1, _N1, _N2, _DT), lambda b, j: (b, 0, 0, j)),
            cspec((2 * _N1, _N1)), cspec((2 * _K2, _N2)),
            cspec((_N1, _N2)), cspec((_N1, _N2)),
        ],
        out_specs=[
            pl.BlockSpec((1, _N1, _N2, _DT), lambda b, j: (b, 0, 0, j)),
            pl.BlockSpec((1, _N1, _N2, _DT), lambda b, j: (b, 0, 0, j)),
        ],
        out_shape=[
            jax.ShapeDtypeStruct((B, _N1, _N2, D), jnp.float32),
            jax.ShapeDtypeStruct((B, _N1, _N2, D), jnp.float32),
        ],
        interpret=interpret,
    )(xr, *consts)
    return out


def kernel(x):
    B, L, D = x.shape
    xr = x.reshape(B, _N1, _N2, D)
    seas, trend = _run(xr)
    return (seas.reshape(B, L, D), trend.reshape(B, L, D))
